# baseline (speedup 1.0000x reference)
import functools

import jax
import jax.numpy as jnp
from jax import lax
from jax.experimental import pallas as pl
from jax.experimental.pallas import tpu as pltpu
from jax.experimental.pallas import tpu_sc as plsc

N = 10000
E = 320000
D = 128
DA = 144

NC, NS = 2, 16
NW = NC * NS
CHUNK = 64
EPW = E // NW
PCHUNK = 8
NPASS = 20
NCHUNK = PCHUNK * NPASS
EPW_PAD = NCHUNK * CHUNK
ROWS_ACC = 10112
RPT = ROWS_ACC // NS
DUMMY = N
DEPTH = 4


def _sc_body(x_hbm, src_hbm, dst_hbm, sum_out,
             acc, idx_src0, idx_src1, idx_dst0, idx_dst1,
             rb0, rb1, rb2, rb3,
             gs0, gs1, gs2, gs3, ss0, ss1, ss2, ss3, rsem):
    c = lax.axis_index("c")
    s = lax.axis_index("s")
    w = c * NS + s
    r0 = s * RPT
    idx_src = (idx_src0, idx_src1)
    idx_dst = (idx_dst0, idx_dst1)
    rowbuf = (rb0, rb1, rb2, rb3)
    gsem = (gs0, gs1, gs2, gs3)
    ssem = (ss0, ss1, ss2, ss3)

    zrow = jnp.zeros((16,), jnp.float32)

    @pl.loop(0, CHUNK)
    def _fill(i):
        for k in range(DA // 16):
            rb0[i, pl.ds(k * 16, 16)] = zrow

    for k in range(9):
        pltpu.sync_copy(rb0, acc.at[pl.ds(r0 + k * CHUNK, CHUNK)])
    pltpu.sync_copy(rb0.at[pl.ds(0, RPT - 9 * CHUNK)],
                    acc.at[pl.ds(r0 + 9 * CHUNK, RPT - 9 * CHUNK)])
    plsc.subcore_barrier()

    r_desc = [
        pltpu.async_copy(src_hbm.at[w, pl.ds(0, PCHUNK)], idx_src[0], rsem),
        pltpu.async_copy(dst_hbm.at[w, pl.ds(0, PCHUNK)], idx_dst[0], rsem),
    ]
    g_desc = [None] * DEPTH
    s_desc = [None] * DEPTH
    pending = [None] * NCHUNK
    t = 0
    for p in range(NPASS):
        hs, hd = idx_src[p % 2], idx_dst[p % 2]
        for d_ in r_desc:
            d_.wait()
        for j in range(PCHUNK):
            if j == 4 and p + 1 < NPASS:
                r_desc = [
                    pltpu.async_copy(
                        src_hbm.at[w, pl.ds((p + 1) * PCHUNK, PCHUNK)],
                        idx_src[(p + 1) % 2], rsem),
                    pltpu.async_copy(
                        dst_hbm.at[w, pl.ds((p + 1) * PCHUNK, PCHUNK)],
                        idx_dst[(p + 1) % 2], rsem),
                ]
            b = t % DEPTH
            if s_desc[b] is not None:
                s_desc[b].wait()
            g_desc[b] = pltpu.async_copy(
                x_hbm.at[hs.at[j]], rowbuf[b], gsem[b])
            pending[t] = (b, hd.at[j])
            tp = t - 2
            if tp >= 0:
                pb, prow = pending[tp]
                g_desc[pb].wait()
                s_desc[pb] = pltpu.async_copy(
                    rowbuf[pb], acc.at[prow], ssem[pb], add=True)
            t += 1
    for tp in (NCHUNK - 2, NCHUNK - 1):
        pb, prow = pending[tp]
        g_desc[pb].wait()
        s_desc[pb] = pltpu.async_copy(
            rowbuf[pb], acc.at[prow], ssem[pb], add=True)
    for b in range(DEPTH):
        if s_desc[b] is not None:
            s_desc[b].wait()

    plsc.subcore_barrier()
    pltpu.sync_copy(acc.at[pl.ds(r0, RPT)], sum_out.at[c, pl.ds(r0, RPT)])


_sc_scatter = functools.partial(
    pl.kernel,
    out_type=[
        jax.ShapeDtypeStruct((NC, ROWS_ACC, DA), jnp.float32),
    ],
    mesh=plsc.VectorSubcoreMesh(core_axis_name="c", subcore_axis_name="s"),
    scratch_types=(
        [pltpu.VMEM_SHARED((ROWS_ACC, DA), jnp.float32)]
        + [pltpu.VMEM((PCHUNK, CHUNK), jnp.int32)] * 4
        + [pltpu.VMEM((CHUNK, DA), jnp.float32)] * DEPTH
        + [pltpu.SemaphoreType.DMA] * (2 * DEPTH + 1)
    ),
    compiler_params=pltpu.CompilerParams(use_tc_tiling_on_sc=False),
)(_sc_body)


def _tc_root_body(x_ref, wr_ref, b_ref, o_ref):
    dn = (((1,), (1,)), ((), ()))
    o_ref[...] = lax.dot_general(
        x_ref[...], wr_ref[...], dn,
        preferred_element_type=jnp.float32) + b_ref[...]


def _tc_agg_body(xr_ref, s0_ref, s1_ref, wl_ref, o_ref):
    cnt = s0_ref[:, D:D + 1] + s1_ref[:, D:D + 1]
    agg = (s0_ref[:, :D] + s1_ref[:, :D]) / jnp.maximum(cnt, 1.0)
    dn = (((1,), (1,)), ((), ()))
    o_ref[...] = lax.dot_general(
        agg, wl_ref[...], dn, preferred_element_type=jnp.float32) + xr_ref[...]


_BLK = 1000
_ROW = pl.BlockSpec((_BLK, D), lambda i: (i, 0))
_AUG = pl.BlockSpec((_BLK, DA), lambda i: (i, 0))
_FULL = pl.BlockSpec((D, D), lambda i: (0, 0))
_BIAS = pl.BlockSpec((1, D), lambda i: (0, 0))


def _tc_root(x, W_r, b_l):
    return pl.pallas_call(
        _tc_root_body,
        grid=(N // _BLK,),
        in_specs=[_ROW, _FULL, _BIAS],
        out_specs=_ROW,
        out_shape=jax.ShapeDtypeStruct((N, D), jnp.float32),
    )(x, W_r, b_l.reshape(1, D))


def _tc_agg(xr, s0, s1, W_l):
    return pl.pallas_call(
        _tc_agg_body,
        grid=(N // _BLK,),
        in_specs=[_ROW, _AUG, _AUG, _FULL],
        out_specs=_ROW,
        out_shape=jax.ShapeDtypeStruct((N, D), jnp.float32),
    )(xr, s0, s1, W_l)


def kernel(x, edge_index, node_type, edge_type, W_l, b_l, W_r):
    x_aug = jnp.concatenate([x, jnp.ones((N, DA - D), jnp.float32)], axis=1)
    src = edge_index[0].reshape(NW, EPW)
    dst = edge_index[1].reshape(NW, EPW)
    pad = EPW_PAD - EPW
    src_p = jnp.concatenate(
        [src, jnp.zeros((NW, pad), jnp.int32)], axis=1).reshape(NW, NCHUNK, CHUNK)
    dst_p = jnp.concatenate(
        [dst, jnp.full((NW, pad), DUMMY, jnp.int32)], axis=1).reshape(NW, NCHUNK, CHUNK)
    xr = _tc_root(x, W_r, b_l)
    (sums,) = _sc_scatter(x_aug, src_p, dst_p)
    return _tc_agg(xr, sums[0, :N], sums[1, :N], W_l)
